# baseline (speedup 1.0000x reference)
import functools

import jax
import jax.numpy as jnp
from jax import lax
from jax.experimental import pallas as pl
from jax.experimental.pallas import tpu as pltpu
from jax.experimental.pallas import tpu_sc as plsc

B = 16384
D = 64
VOCAB = 1000000
NC = 2
NS = 16
NW = NC * NS
BPW = B // NW
L = 16
CW = 384
NG = B // L
RING = 16
POSB = 14
SL = 4096
TW = 128

_params = pltpu.CompilerParams(needs_layout_passes=False)
_mesh = plsc.VectorSubcoreMesh(core_axis_name="c", subcore_axis_name="s")


def _extract_stage(u_pos, v, u_tabT, v_tabT, u_tailT, v_tailT):
    @functools.partial(
        pl.kernel,
        out_type=(jax.ShapeDtypeStruct((B * D,), jnp.float32),
                  jax.ShapeDtypeStruct((B * D,), jnp.float32)),
        mesh=_mesh,
        compiler_params=_params,
        scratch_types=[
            pltpu.VMEM((SL,), jnp.int32),
            pltpu.VMEM((B + L,), jnp.int32),
            pltpu.VMEM((4, D, CW), jnp.float32),
            pltpu.VMEM((D, TW), jnp.float32),
            pltpu.VMEM((2 * L,), jnp.int32),
            pltpu.VMEM((RING, D), jnp.float32),
            pltpu.SemaphoreType.DMA,
            pltpu.SemaphoreType.DMA,
            pltpu.SemaphoreType.DMA,
            pltpu.SemaphoreType.DMA,
            pltpu.SemaphoreType.DMA,
        ],
    )
    def k(u_pos_hbm, v_hbm, u_tab_hbm, v_tab_hbm, u_tail_hbm, v_tail_hbm,
          ustage_hbm, vstage_hbm,
          idx_v, pk_v, chunk_v, tail_v, hq_v, ring_v,
          sem_c0, sem_c1, sem_c2, sem_c3, sem_r):
        wid = lax.axis_index("s") * NC + lax.axis_index("c")
        lo = ((wid * (VOCAB // NW)) // CW) * CW
        hi_full = jnp.where(wid == NW - 1, (VOCAB // CW) * CW,
                            (((wid + 1) * (VOCAB // NW)) // CW) * CW)
        hi = jnp.where(wid == NW - 1, VOCAB, hi_full)
        lo = pl.multiple_of(lo, CW)
        nchunks = (hi_full - lo) // CW
        lanes = lax.iota(jnp.int32, L)

        def table_pass(idx_hbm, tab_hbm, tail_hbm, stage_hbm):
            def section(sec, n):
                pltpu.sync_copy(idx_hbm.at[pl.ds(sec * SL, SL)], idx_v)

                def compress(g, nn):
                    r = idx_v[pl.ds(g * L, L)]
                    m = (r >= lo) & (r < hi)
                    packed = ((r - lo) << POSB) | (sec * SL + g * L + lanes)
                    plsc.store_compressed(pk_v.at[pl.ds(nn, L)], packed,
                                          mask=m)
                    cnt = plsc.all_reduce_population_count(m)[0]
                    return nn + cnt

                return lax.fori_loop(0, SL // L, compress, n)

            n_mine = lax.fori_loop(0, B // SL, section, 0)

            def chunk_start(kk):
                return pl.multiple_of(lo + kk * CW, CW)

            sems = [sem_c0, sem_c1, sem_c2, sem_c3]

            def fire(kk, slot):
                pltpu.async_copy(
                    tab_hbm.at[:, pl.ds(chunk_start(kk), CW)],
                    chunk_v.at[slot], sems[slot])

            def extract_span(cbuf, start, own_lo, own_hi, dma_in):
                def per_group(g, dma_cnt):
                    p = pk_v[pl.ds(g * L, L)]
                    r = (p >> POSB) + lo
                    m = (lanes < (n_mine - g * L)) & (r >= own_lo) & (r < own_hi)
                    plsc.store_compressed(hq_v.at[pl.ds(0, L)], p, mask=m)
                    nhit = plsc.all_reduce_population_count(m)[0]

                    def per_hit(e, dc):
                        pe = hq_v[pl.ds(e, L)][0]
                        j = (pe >> POSB) + lo - start
                        pos = pe & ((1 << POSB) - 1)
                        slot_r = lax.rem(dc, RING)

                        @pl.when((slot_r == 0) & (dc > 0))
                        def _():
                            for _ in range(RING):
                                pltpu.make_async_copy(
                                    ring_v.at[0],
                                    stage_hbm.at[pl.ds(0, D)], sem_r).wait()

                        jv = lanes * 0 + j
                        for f in range(D // L):
                            ring_v[slot_r, pl.ds(f * L, L)] = (
                                plsc.load_gather(cbuf, [f * L + lanes, jv]))
                        pltpu.async_copy(
                            ring_v.at[slot_r],
                            stage_hbm.at[pl.ds(pos * D, D)], sem_r)
                        return dc + 1

                    return lax.fori_loop(0, nhit, per_hit, dma_cnt)

                ngrp = (n_mine + L - 1) // L
                return lax.fori_loop(0, ngrp, per_group, dma_in)

            def process(kk, slot, dma_in):
                start = chunk_start(kk)
                pltpu.make_async_copy(
                    tab_hbm.at[:, pl.ds(start, CW)],
                    chunk_v.at[slot], sems[slot]).wait()
                return extract_span(chunk_v.at[slot], start,
                                    start, start + CW, dma_in)

            fire(0, 0)
            fire(1, 1)
            fire(2, 2)

            def per_quad(p, carry):
                for o in range(4):
                    kk = 4 * p + o

                    @pl.when(kk + 3 < nchunks)
                    def _(kk=kk, o=o):
                        fire(kk + 3, (o + 3) % 4)

                    carry = lax.cond(
                        kk < nchunks,
                        lambda c, kk=kk, o=o: process(kk, o, c),
                        lambda c: c, carry)
                return carry

            nquad = (nchunks + 3) // 4
            total_dma = lax.fori_loop(0, nquad, per_quad, 0)

            def tail(c):
                pltpu.sync_copy(tail_hbm, tail_v)
                return extract_span(tail_v, VOCAB - TW, hi_full, hi, c)

            total_dma = lax.cond(hi > hi_full, tail, lambda c: c, total_dma)

            rem = jnp.where(
                total_dma > 0,
                total_dma - ((total_dma - 1) // RING) * RING, 0)

            def drain(e, carry):
                @pl.when(e < rem)
                def _():
                    pltpu.make_async_copy(
                        ring_v.at[0], stage_hbm.at[pl.ds(0, D)], sem_r).wait()
                return carry

            lax.fori_loop(0, RING, drain, 0)

        table_pass(u_pos_hbm, u_tab_hbm, u_tail_hbm, ustage_hbm)
        table_pass(v_hbm, v_tab_hbm, v_tail_hbm, vstage_hbm)

    return k(u_pos, v, u_tabT, v_tabT, u_tailT, v_tailT)


def _dot_stage(ustage, vstage):
    @functools.partial(
        pl.kernel,
        out_type=jax.ShapeDtypeStruct((B,), jnp.float32),
        mesh=_mesh,
        compiler_params=_params,
        scratch_types=[
            pltpu.VMEM((BPW * D,), jnp.float32),
            pltpu.VMEM((BPW * D,), jnp.float32),
            pltpu.VMEM((BPW,), jnp.float32),
            pltpu.VMEM((L * L,), jnp.float32),
            pltpu.SemaphoreType.DMA,
            pltpu.SemaphoreType.DMA,
        ],
    )
    def k(ustage_hbm, vstage_hbm, out_hbm, urows_v, vrows_v, out_v, tbuf_v,
          sem_u, sem_v):
        wid = lax.axis_index("s") * NC + lax.axis_index("c")
        base = wid * BPW
        cu = pltpu.async_copy(
            ustage_hbm.at[pl.ds(base * D, BPW * D)], urows_v, sem_u)
        cv = pltpu.async_copy(
            vstage_hbm.at[pl.ds(base * D, BPW * D)], vrows_v, sem_v)
        cu.wait()
        cv.wait()
        lanes = lax.iota(jnp.int32, L)

        def group(g, carry):
            base_r = g * L
            for r in range(L):
                s = jnp.zeros((L,), jnp.float32)
                for j in range(D // L):
                    uu = urows_v[pl.ds((base_r + r) * D + j * L, L)]
                    vv = vrows_v[pl.ds((base_r + r) * D + j * L, L)]
                    s = s + uu * vv
                plsc.store_scatter(tbuf_v, [lanes * L + r], s)
            acc = jnp.zeros((L,), jnp.float32)
            for kk in range(L):
                acc = acc + tbuf_v[pl.ds(kk * L, L)]
            out_v[pl.ds(g * L, L)] = acc
            return carry

        lax.fori_loop(0, BPW // L, group, 0)
        pltpu.sync_copy(out_v, out_hbm.at[pl.ds(base, BPW)])

    return k(ustage, vstage)


def _logsigmoid_tc(scores):
    x = scores.reshape(B // 128, 128)

    def body(x_ref, o_ref):
        o_ref[...] = jax.nn.log_sigmoid(x_ref[...])

    y = pl.pallas_call(
        body,
        out_shape=jax.ShapeDtypeStruct((B // 128, 128), jnp.float32),
    )(x)
    return y.reshape(B)


def kernel(u_pos, v, u_table, v_table):
    u_tailT = u_table[VOCAB - TW:, :].T
    v_tailT = v_table[VOCAB - TW:, :].T
    ustage, vstage = _extract_stage(u_pos, v, u_table.T, v_table.T,
                                    u_tailT, v_tailT)
    scores = _dot_stage(ustage, vstage)
    return _logsigmoid_tc(scores)


# speedup vs baseline: 1.1253x; 1.1253x over previous
import functools

import jax
import jax.numpy as jnp
from jax import lax
from jax.experimental import pallas as pl
from jax.experimental.pallas import tpu as pltpu
from jax.experimental.pallas import tpu_sc as plsc

B = 16384
D = 64
VOCAB = 1000000
NC = 2
NS = 16
NW = NC * NS
BPW = B // NW
L = 16
CW = 512
NG = B // L
RING = 16
POSB = 14
SL = 4096
TW = 128

_params = pltpu.CompilerParams(needs_layout_passes=False)
_mesh = plsc.VectorSubcoreMesh(core_axis_name="c", subcore_axis_name="s")


def _extract_stage(u_pos, v, u_tabT, v_tabT, u_tailT, v_tailT):
    @functools.partial(
        pl.kernel,
        out_type=(jax.ShapeDtypeStruct((B * D,), jnp.float32),
                  jax.ShapeDtypeStruct((B * D,), jnp.float32)),
        mesh=_mesh,
        compiler_params=_params,
        scratch_types=[
            pltpu.VMEM((SL,), jnp.int32),
            pltpu.VMEM((B + L,), jnp.int32),
            pltpu.VMEM((3, D, CW), jnp.float32),
            pltpu.VMEM((D, TW), jnp.float32),
            pltpu.VMEM((2 * L,), jnp.int32),
            pltpu.VMEM((RING, D), jnp.float32),
            pltpu.SemaphoreType.DMA,
            pltpu.SemaphoreType.DMA,
            pltpu.SemaphoreType.DMA,
            pltpu.SemaphoreType.DMA,
        ],
    )
    def k(u_pos_hbm, v_hbm, u_tab_hbm, v_tab_hbm, u_tail_hbm, v_tail_hbm,
          ustage_hbm, vstage_hbm,
          idx_v, pk_v, chunk_v, tail_v, hq_v, ring_v,
          sem_c0, sem_c1, sem_c2, sem_r):
        wid = lax.axis_index("s") * NC + lax.axis_index("c")
        lo = ((wid * (VOCAB // NW)) // CW) * CW
        hi_full = jnp.where(wid == NW - 1, (VOCAB // CW) * CW,
                            (((wid + 1) * (VOCAB // NW)) // CW) * CW)
        hi = jnp.where(wid == NW - 1, VOCAB, hi_full)
        lo = pl.multiple_of(lo, CW)
        nchunks = (hi_full - lo) // CW
        lanes = lax.iota(jnp.int32, L)

        def table_pass(idx_hbm, tab_hbm, tail_hbm, stage_hbm):
            def section(sec, n):
                pltpu.sync_copy(idx_hbm.at[pl.ds(sec * SL, SL)], idx_v)

                def compress(g, nn):
                    r = idx_v[pl.ds(g * L, L)]
                    m = (r >= lo) & (r < hi)
                    packed = ((r - lo) << POSB) | (sec * SL + g * L + lanes)
                    plsc.store_compressed(pk_v.at[pl.ds(nn, L)], packed,
                                          mask=m)
                    cnt = plsc.all_reduce_population_count(m)[0]
                    return nn + cnt

                return lax.fori_loop(0, SL // L, compress, n)

            n_mine = lax.fori_loop(0, B // SL, section, 0)

            def chunk_start(kk):
                return pl.multiple_of(lo + kk * CW, CW)

            sems = [sem_c0, sem_c1, sem_c2]

            def fire(kk, slot):
                pltpu.async_copy(
                    tab_hbm.at[:, pl.ds(chunk_start(kk), CW)],
                    chunk_v.at[slot], sems[slot])

            def extract_span(cbuf, start, own_lo, own_hi, dma_in):
                def per_group(g, dma_cnt):
                    p = pk_v[pl.ds(g * L, L)]
                    r = (p >> POSB) + lo
                    m = (lanes < (n_mine - g * L)) & (r >= own_lo) & (r < own_hi)
                    plsc.store_compressed(hq_v.at[pl.ds(0, L)], p, mask=m)
                    nhit = plsc.all_reduce_population_count(m)[0]

                    def per_hit(e, dc):
                        pe = hq_v[pl.ds(e, L)][0]
                        j = (pe >> POSB) + lo - start
                        pos = pe & ((1 << POSB) - 1)
                        slot_r = lax.rem(dc, RING)

                        @pl.when((slot_r == 0) & (dc > 0))
                        def _():
                            for _ in range(RING):
                                pltpu.make_async_copy(
                                    ring_v.at[0],
                                    stage_hbm.at[pl.ds(0, D)], sem_r).wait()

                        jv = lanes * 0 + j
                        for f in range(D // L):
                            ring_v[slot_r, pl.ds(f * L, L)] = (
                                plsc.load_gather(cbuf, [f * L + lanes, jv]))
                        pltpu.async_copy(
                            ring_v.at[slot_r],
                            stage_hbm.at[pl.ds(pos * D, D)], sem_r)
                        return dc + 1

                    return lax.fori_loop(0, nhit, per_hit, dma_cnt)

                ngrp = (n_mine + L - 1) // L
                return lax.fori_loop(0, ngrp, per_group, dma_in)

            def process(kk, slot, dma_in):
                start = chunk_start(kk)
                pltpu.make_async_copy(
                    tab_hbm.at[:, pl.ds(start, CW)],
                    chunk_v.at[slot], sems[slot]).wait()
                return extract_span(chunk_v.at[slot], start,
                                    start, start + CW, dma_in)

            fire(0, 0)
            fire(1, 1)

            def per_triple(p, carry):
                for o in range(3):
                    kk = 3 * p + o

                    @pl.when(kk + 2 < nchunks)
                    def _(kk=kk, o=o):
                        fire(kk + 2, (o + 2) % 3)

                    carry = lax.cond(
                        kk < nchunks,
                        lambda c, kk=kk, o=o: process(kk, o, c),
                        lambda c: c, carry)
                return carry

            ntrip = (nchunks + 2) // 3
            total_dma = lax.fori_loop(0, ntrip, per_triple, 0)

            def tail(c):
                pltpu.sync_copy(tail_hbm, tail_v)
                return extract_span(tail_v, VOCAB - TW, hi_full, hi, c)

            total_dma = lax.cond(hi > hi_full, tail, lambda c: c, total_dma)

            rem = jnp.where(
                total_dma > 0,
                total_dma - ((total_dma - 1) // RING) * RING, 0)

            def drain(e, carry):
                @pl.when(e < rem)
                def _():
                    pltpu.make_async_copy(
                        ring_v.at[0], stage_hbm.at[pl.ds(0, D)], sem_r).wait()
                return carry

            lax.fori_loop(0, RING, drain, 0)

        table_pass(u_pos_hbm, u_tab_hbm, u_tail_hbm, ustage_hbm)
        table_pass(v_hbm, v_tab_hbm, v_tail_hbm, vstage_hbm)

    return k(u_pos, v, u_tabT, v_tabT, u_tailT, v_tailT)


def _dot_stage(ustage, vstage):
    @functools.partial(
        pl.kernel,
        out_type=jax.ShapeDtypeStruct((B,), jnp.float32),
        mesh=_mesh,
        compiler_params=_params,
        scratch_types=[
            pltpu.VMEM((BPW * D,), jnp.float32),
            pltpu.VMEM((BPW * D,), jnp.float32),
            pltpu.VMEM((BPW,), jnp.float32),
            pltpu.VMEM((L * L,), jnp.float32),
            pltpu.SemaphoreType.DMA,
            pltpu.SemaphoreType.DMA,
        ],
    )
    def k(ustage_hbm, vstage_hbm, out_hbm, urows_v, vrows_v, out_v, tbuf_v,
          sem_u, sem_v):
        wid = lax.axis_index("s") * NC + lax.axis_index("c")
        base = wid * BPW
        cu = pltpu.async_copy(
            ustage_hbm.at[pl.ds(base * D, BPW * D)], urows_v, sem_u)
        cv = pltpu.async_copy(
            vstage_hbm.at[pl.ds(base * D, BPW * D)], vrows_v, sem_v)
        cu.wait()
        cv.wait()
        lanes = lax.iota(jnp.int32, L)

        def group(g, carry):
            base_r = g * L
            for r in range(L):
                s = jnp.zeros((L,), jnp.float32)
                for j in range(D // L):
                    uu = urows_v[pl.ds((base_r + r) * D + j * L, L)]
                    vv = vrows_v[pl.ds((base_r + r) * D + j * L, L)]
                    s = s + uu * vv
                plsc.store_scatter(tbuf_v, [lanes * L + r], s)
            acc = jnp.zeros((L,), jnp.float32)
            for kk in range(L):
                acc = acc + tbuf_v[pl.ds(kk * L, L)]
            out_v[pl.ds(g * L, L)] = acc
            return carry

        lax.fori_loop(0, BPW // L, group, 0)
        pltpu.sync_copy(out_v, out_hbm.at[pl.ds(base, BPW)])

    return k(ustage, vstage)


def _logsigmoid_tc(scores):
    x = scores.reshape(B // 128, 128)

    def body(x_ref, o_ref):
        o_ref[...] = jax.nn.log_sigmoid(x_ref[...])

    y = pl.pallas_call(
        body,
        out_shape=jax.ShapeDtypeStruct((B // 128, 128), jnp.float32),
    )(x)
    return y.reshape(B)


def kernel(u_pos, v, u_table, v_table):
    u_tailT = u_table[VOCAB - TW:, :].T
    v_tailT = v_table[VOCAB - TW:, :].T
    ustage, vstage = _extract_stage(u_pos, v, u_table.T, v_table.T,
                                    u_tailT, v_tailT)
    scores = _dot_stage(ustage, vstage)
    return _logsigmoid_tc(scores)


# speedup vs baseline: 1.2125x; 1.0775x over previous
import functools

import jax
import jax.numpy as jnp
from jax import lax
from jax.experimental import pallas as pl
from jax.experimental.pallas import tpu as pltpu
from jax.experimental.pallas import tpu_sc as plsc

B = 16384
D = 64
VOCAB = 1000000
NC = 2
NS = 16
NW = NC * NS
BPW = B // NW
L = 16
CW = 512
NG = B // L
RING = 8
POSB = 14
SL = 4096
TW = 128

_params = pltpu.CompilerParams(needs_layout_passes=False)
_mesh = plsc.VectorSubcoreMesh(core_axis_name="c", subcore_axis_name="s")


def _extract_stage(u_pos, v, u_tabT, v_tabT, u_tailT, v_tailT):
    @functools.partial(
        pl.kernel,
        out_type=(jax.ShapeDtypeStruct((B * D,), jnp.float32),
                  jax.ShapeDtypeStruct((B * D,), jnp.float32)),
        mesh=_mesh,
        compiler_params=_params,
        scratch_types=[
            pltpu.VMEM((SL,), jnp.int32),
            pltpu.VMEM((B + L,), jnp.int32),
            pltpu.VMEM((3, D, CW), jnp.float32),
            pltpu.VMEM((D, TW), jnp.float32),
            pltpu.VMEM((2 * L,), jnp.int32),
            pltpu.VMEM((RING, D), jnp.float32),
            pltpu.SemaphoreType.DMA,
            pltpu.SemaphoreType.DMA,
            pltpu.SemaphoreType.DMA,
            pltpu.SemaphoreType.DMA,
        ],
    )
    def k(u_pos_hbm, v_hbm, u_tab_hbm, v_tab_hbm, u_tail_hbm, v_tail_hbm,
          ustage_hbm, vstage_hbm,
          idx_v, pk_v, chunk_v, tail_v, hq_v, ring_v,
          sem_c0, sem_c1, sem_c2, sem_r):
        wid = lax.axis_index("s") * NC + lax.axis_index("c")
        lo = ((wid * (VOCAB // NW)) // CW) * CW
        hi_full = jnp.where(wid == NW - 1, (VOCAB // CW) * CW,
                            (((wid + 1) * (VOCAB // NW)) // CW) * CW)
        hi = jnp.where(wid == NW - 1, VOCAB, hi_full)
        lo = pl.multiple_of(lo, CW)
        nchunks = (hi_full - lo) // CW
        lanes = lax.iota(jnp.int32, L)

        def table_pass(idx_hbm, tab_hbm, tail_hbm, stage_hbm):
            def section(sec, n):
                pltpu.sync_copy(idx_hbm.at[pl.ds(sec * SL, SL)], idx_v)

                def compress(g, nn):
                    r = idx_v[pl.ds(g * L, L)]
                    m = (r >= lo) & (r < hi)
                    packed = ((r - lo) << POSB) | (sec * SL + g * L + lanes)
                    plsc.store_compressed(pk_v.at[pl.ds(nn, L)], packed,
                                          mask=m)
                    cnt = plsc.all_reduce_population_count(m)[0]
                    return nn + cnt

                return lax.fori_loop(0, SL // L, compress, n)

            n_mine = lax.fori_loop(0, B // SL, section, 0)

            def chunk_start(kk):
                return pl.multiple_of(lo + kk * CW, CW)

            sems = [sem_c0, sem_c1, sem_c2]

            def fire(kk, slot):
                pltpu.async_copy(
                    tab_hbm.at[:, pl.ds(chunk_start(kk), CW)],
                    chunk_v.at[slot], sems[slot])

            def extract_span(cbuf, start, own_lo, own_hi, dma_in):
                def per_group(g, dma_cnt):
                    p = pk_v[pl.ds(g * L, L)]
                    r = (p >> POSB) + lo
                    m = (lanes < (n_mine - g * L)) & (r >= own_lo) & (r < own_hi)
                    plsc.store_compressed(hq_v.at[pl.ds(0, L)], p, mask=m)
                    nhit = plsc.all_reduce_population_count(m)[0]

                    def per_hit(e, dc):
                        pe = hq_v[pl.ds(e, L)][0]
                        j = (pe >> POSB) + lo - start
                        pos = pe & ((1 << POSB) - 1)
                        slot_r = lax.rem(dc, RING)

                        @pl.when((slot_r == 0) & (dc > 0))
                        def _():
                            for _ in range(RING):
                                pltpu.make_async_copy(
                                    ring_v.at[0],
                                    stage_hbm.at[pl.ds(0, D)], sem_r).wait()

                        jv = lanes * 0 + j
                        for f in range(D // L):
                            ring_v[slot_r, pl.ds(f * L, L)] = (
                                plsc.load_gather(cbuf, [f * L + lanes, jv]))
                        pltpu.async_copy(
                            ring_v.at[slot_r],
                            stage_hbm.at[pl.ds(pos * D, D)], sem_r)
                        return dc + 1

                    return lax.fori_loop(0, nhit, per_hit, dma_cnt)

                ngrp = (n_mine + L - 1) // L
                return lax.fori_loop(0, ngrp, per_group, dma_in)

            def process(kk, slot, dma_in):
                start = chunk_start(kk)
                pltpu.make_async_copy(
                    tab_hbm.at[:, pl.ds(start, CW)],
                    chunk_v.at[slot], sems[slot]).wait()
                return extract_span(chunk_v.at[slot], start,
                                    start, start + CW, dma_in)

            fire(0, 0)
            fire(1, 1)

            def per_triple(p, carry):
                for o in range(3):
                    kk = 3 * p + o

                    @pl.when(kk + 2 < nchunks)
                    def _(kk=kk, o=o):
                        fire(kk + 2, (o + 2) % 3)

                    carry = lax.cond(
                        kk < nchunks,
                        lambda c, kk=kk, o=o: process(kk, o, c),
                        lambda c: c, carry)
                return carry

            ntrip = (nchunks + 2) // 3
            total_dma = lax.fori_loop(0, ntrip, per_triple, 0)

            def tail(c):
                pltpu.sync_copy(tail_hbm, tail_v)
                return extract_span(tail_v, VOCAB - TW, hi_full, hi, c)

            total_dma = lax.cond(hi > hi_full, tail, lambda c: c, total_dma)

            rem = jnp.where(
                total_dma > 0,
                total_dma - ((total_dma - 1) // RING) * RING, 0)

            def drain(e, carry):
                @pl.when(e < rem)
                def _():
                    pltpu.make_async_copy(
                        ring_v.at[0], stage_hbm.at[pl.ds(0, D)], sem_r).wait()
                return carry

            lax.fori_loop(0, RING, drain, 0)

        table_pass(u_pos_hbm, u_tab_hbm, u_tail_hbm, ustage_hbm)
        table_pass(v_hbm, v_tab_hbm, v_tail_hbm, vstage_hbm)

    return k(u_pos, v, u_tabT, v_tabT, u_tailT, v_tailT)


def _dot_stage(ustage, vstage):
    @functools.partial(
        pl.kernel,
        out_type=jax.ShapeDtypeStruct((B,), jnp.float32),
        mesh=_mesh,
        compiler_params=_params,
        scratch_types=[
            pltpu.VMEM((BPW * D,), jnp.float32),
            pltpu.VMEM((BPW * D,), jnp.float32),
            pltpu.VMEM((BPW,), jnp.float32),
            pltpu.VMEM((L * L,), jnp.float32),
            pltpu.SemaphoreType.DMA,
            pltpu.SemaphoreType.DMA,
        ],
    )
    def k(ustage_hbm, vstage_hbm, out_hbm, urows_v, vrows_v, out_v, tbuf_v,
          sem_u, sem_v):
        wid = lax.axis_index("s") * NC + lax.axis_index("c")
        base = wid * BPW
        cu = pltpu.async_copy(
            ustage_hbm.at[pl.ds(base * D, BPW * D)], urows_v, sem_u)
        cv = pltpu.async_copy(
            vstage_hbm.at[pl.ds(base * D, BPW * D)], vrows_v, sem_v)
        cu.wait()
        cv.wait()
        lanes = lax.iota(jnp.int32, L)

        def group(g, carry):
            base_r = g * L
            for r in range(L):
                s = jnp.zeros((L,), jnp.float32)
                for j in range(D // L):
                    uu = urows_v[pl.ds((base_r + r) * D + j * L, L)]
                    vv = vrows_v[pl.ds((base_r + r) * D + j * L, L)]
                    s = s + uu * vv
                plsc.store_scatter(tbuf_v, [lanes * L + r], s)
            acc = jnp.zeros((L,), jnp.float32)
            for kk in range(L):
                acc = acc + tbuf_v[pl.ds(kk * L, L)]
            out_v[pl.ds(g * L, L)] = acc
            return carry

        lax.fori_loop(0, BPW // L, group, 0)
        pltpu.sync_copy(out_v, out_hbm.at[pl.ds(base, BPW)])

    return k(ustage, vstage)


def _logsigmoid_tc(scores):
    x = scores.reshape(B // 128, 128)

    def body(x_ref, o_ref):
        o_ref[...] = jax.nn.log_sigmoid(x_ref[...])

    y = pl.pallas_call(
        body,
        out_shape=jax.ShapeDtypeStruct((B // 128, 128), jnp.float32),
    )(x)
    return y.reshape(B)


def kernel(u_pos, v, u_table, v_table):
    u_tailT = u_table[VOCAB - TW:, :].T
    v_tailT = v_table[VOCAB - TW:, :].T
    ustage, vstage = _extract_stage(u_pos, v, u_table.T, v_table.T,
                                    u_tailT, v_tailT)
    scores = _dot_stage(ustage, vstage)
    return _logsigmoid_tc(scores)
